# initial kernel scaffold (unmeasured)
import jax
import jax.numpy as jnp
from jax import lax
from jax.experimental import pallas as pl
from jax.experimental.pallas import tpu as pltpu

N_DEV = 4
SQ_LOC = 256
D_MODEL = 1024
SKV = 4096
H_LOC = 8
DH = 128
SQ = SQ_LOC * N_DEV
D_HEADS = H_LOC * DH
SCALE = 0.08838834764831843
QBLK = 256
N_QBLK = SQ // QBLK


def kernel(x, Wq, K_ext, V_ext, Wo):
    x2 = x.reshape(SQ_LOC, D_MODEL)

    def body(x_ref, wq_ref, k_hbm, v_hbm, wo_ref, out_ref,
             xg, comm, k_loc, v_loc, ctx, part, rs_send, rs_recv,
             kv_sems, ag_send_sems, ag_recv_sems, rs_send_sems, rs_recv_sems):
        my = lax.axis_index("i")
        left = lax.rem(my + N_DEV - 1, N_DEV)
        right = lax.rem(my + 1, N_DEV)
        h0 = my * H_LOC

        barrier = pltpu.get_barrier_semaphore()
        for nbr in (left, right):
            pl.semaphore_signal(barrier, inc=1, device_id=(nbr,),
                                device_id_type=pl.DeviceIdType.MESH)
        pl.semaphore_wait(barrier, 2)

        k_copy = pltpu.make_async_copy(
            k_hbm.at[0, :, pl.ds(h0, H_LOC), :], k_loc, kv_sems.at[0])
        v_copy = pltpu.make_async_copy(
            v_hbm.at[0, :, pl.ds(h0, H_LOC), :], v_loc, kv_sems.at[1])
        k_copy.start()
        v_copy.start()

        xg[pl.ds(my * SQ_LOC, SQ_LOC), :] = x_ref[...]
        comm[0] = x_ref[...]
        for hop in range(N_DEV - 1):
            rdma = pltpu.make_async_remote_copy(
                src_ref=comm.at[hop],
                dst_ref=comm.at[hop + 1],
                send_sem=ag_send_sems.at[hop],
                recv_sem=ag_recv_sems.at[hop],
                device_id=(right,),
                device_id_type=pl.DeviceIdType.MESH,
            )
            rdma.start()
            rdma.wait()
            origin = lax.rem(my + N_DEV - 1 - hop, N_DEV)
            xg[pl.ds(origin * SQ_LOC, SQ_LOC), :] = comm[hop + 1]

        k_copy.wait()
        v_copy.wait()

        q = lax.dot_general(xg[...], wq_ref[...], (((1,), (0,)), ((), ())),
                            preferred_element_type=jnp.float32)

        for qb in range(N_QBLK):
            r0 = qb * QBLK
            row = lax.broadcasted_iota(jnp.int32, (QBLK, SKV), 0) + r0
            col = lax.broadcasted_iota(jnp.int32, (QBLK, SKV), 1)
            qblk_id = row // 64
            kblk_id = col // 64
            mask = ((qblk_id == kblk_id) | (kblk_id == 0)
                    | (lax.rem(qblk_id + kblk_id, 3) == 0))
            bias = jnp.where(mask, 0.0, -1e9).astype(jnp.float32)
            for h in range(H_LOC):
                qh = q[r0:r0 + QBLK, h * DH:(h + 1) * DH]
                s = lax.dot_general(qh, k_loc[:, h, :],
                                    (((1,), (1,)), ((), ())),
                                    preferred_element_type=jnp.float32)
                s = s * SCALE + bias
                m = jnp.max(s, axis=1, keepdims=True)
                w = jnp.exp(s - m)
                den = jnp.sum(w, axis=1, keepdims=True)
                c = lax.dot_general(w, v_loc[:, h, :],
                                    (((1,), (0,)), ((), ())),
                                    preferred_element_type=jnp.float32)
                ctx[pl.ds(r0, QBLK), pl.ds(h * DH, DH)] = c / den

        part[...] = lax.dot_general(ctx[...], wo_ref[...],
                                    (((1,), (0,)), ((), ())),
                                    preferred_element_type=jnp.float32)

        for s in range(N_DEV - 1):
            send_chunk = lax.rem(my + N_DEV - 1 - s, N_DEV)
            if s == 0:
                src = part.at[pl.ds(send_chunk * SQ_LOC, SQ_LOC), :]
            else:
                src = rs_send.at[s - 1]
            rdma = pltpu.make_async_remote_copy(
                src_ref=src,
                dst_ref=rs_recv.at[s],
                send_sem=rs_send_sems.at[s],
                recv_sem=rs_recv_sems.at[s],
                device_id=(right,),
                device_id_type=pl.DeviceIdType.MESH,
            )
            rdma.start()
            rdma.wait()
            recv_chunk = lax.rem(my + N_DEV - 2 - s, N_DEV)
            if s < N_DEV - 2:
                rs_send[s] = (rs_recv[s]
                              + part[pl.ds(recv_chunk * SQ_LOC, SQ_LOC), :])
            else:
                out_ref[0] = (rs_recv[s]
                              + part[pl.ds(my * SQ_LOC, SQ_LOC), :])

    return pl.pallas_call(
        body,
        out_shape=jax.ShapeDtypeStruct((1, SQ_LOC, D_MODEL), jnp.float32),
        in_specs=[
            pl.BlockSpec(memory_space=pltpu.VMEM),
            pl.BlockSpec(memory_space=pltpu.VMEM),
            pl.BlockSpec(memory_space=pltpu.ANY),
            pl.BlockSpec(memory_space=pltpu.ANY),
            pl.BlockSpec(memory_space=pltpu.VMEM),
        ],
        out_specs=pl.BlockSpec(memory_space=pltpu.VMEM),
        scratch_shapes=[
            pltpu.VMEM((SQ, D_MODEL), jnp.float32),
            pltpu.VMEM((N_DEV, SQ_LOC, D_MODEL), jnp.float32),
            pltpu.VMEM((SKV, H_LOC, DH), jnp.float32),
            pltpu.VMEM((SKV, H_LOC, DH), jnp.float32),
            pltpu.VMEM((SQ, D_HEADS), jnp.float32),
            pltpu.VMEM((SQ, D_MODEL), jnp.float32),
            pltpu.VMEM((N_DEV - 2, SQ_LOC, D_MODEL), jnp.float32),
            pltpu.VMEM((N_DEV - 1, SQ_LOC, D_MODEL), jnp.float32),
            pltpu.SemaphoreType.DMA((2,)),
            pltpu.SemaphoreType.DMA((N_DEV - 1,)),
            pltpu.SemaphoreType.DMA((N_DEV - 1,)),
            pltpu.SemaphoreType.DMA((N_DEV - 1,)),
            pltpu.SemaphoreType.DMA((N_DEV - 1,)),
        ],
        compiler_params=pltpu.CompilerParams(
            collective_id=0,
            vmem_limit_bytes=128 * 1024 * 1024,
        ),
    )(x2, Wq, K_ext, V_ext, Wo)


# baseline (device time: 252358 ns/iter reference)
import jax
import jax.numpy as jnp
from jax import lax
from jax.experimental import pallas as pl
from jax.experimental.pallas import tpu as pltpu

N_DEV = 4
SQ_LOC = 256
D_MODEL = 1024
SKV = 4096
H_LOC = 8
DH = 128
SQ = SQ_LOC * N_DEV
D_HEADS = H_LOC * DH
SCALE = 0.08838834764831843
QBLK = 256
N_QBLK = SQ // QBLK


def kernel(x, Wq, K_ext, V_ext, Wo):
    x2 = x.reshape(SQ_LOC, D_MODEL)

    def body(x_ref, wq_ref, k_hbm, v_hbm, wo_ref, out_ref,
             comm, q_ref, kbuf, vbuf, ctx, part, rs_send, rs_recv,
             kv_sems, ag_send_sems, ag_recv_sems, rs_send_sems, rs_recv_sems):
        my = lax.axis_index("i")
        left = lax.rem(my + N_DEV - 1, N_DEV)
        right = lax.rem(my + 1, N_DEV)
        h0 = my * H_LOC

        barrier = pltpu.get_barrier_semaphore()
        for nbr in (left, right):
            pl.semaphore_signal(barrier, inc=1, device_id=(nbr,),
                                device_id_type=pl.DeviceIdType.MESH)
        pl.semaphore_wait(barrier, 2)

        comm[0] = x_ref[...]
        q_ref[pl.ds(my * SQ_LOC, SQ_LOC), :] = lax.dot_general(
            x_ref[...], wq_ref[...], (((1,), (0,)), ((), ())),
            preferred_element_type=jnp.float32)
        for hop in range(N_DEV - 1):
            rdma = pltpu.make_async_remote_copy(
                src_ref=comm.at[hop],
                dst_ref=comm.at[hop + 1],
                send_sem=ag_send_sems.at[hop],
                recv_sem=ag_recv_sems.at[hop],
                device_id=(right,),
                device_id_type=pl.DeviceIdType.MESH,
            )
            rdma.start()
            rdma.wait()
            origin = lax.rem(my + N_DEV - 1 - hop, N_DEV)
            q_ref[pl.ds(origin * SQ_LOC, SQ_LOC), :] = lax.dot_general(
                comm[hop + 1], wq_ref[...], (((1,), (0,)), ((), ())),
                preferred_element_type=jnp.float32)

        def attn_step(i, carry):
            h = i // N_QBLK
            qb = lax.rem(i, N_QBLK)

            @pl.when(qb == 0)
            def _():
                kc = pltpu.make_async_copy(
                    k_hbm.at[0, :, h0 + h, :], kbuf, kv_sems.at[0])
                vc = pltpu.make_async_copy(
                    v_hbm.at[0, :, h0 + h, :], vbuf, kv_sems.at[1])
                kc.start()
                vc.start()
                kc.wait()
                vc.wait()

            r0 = qb * QBLK
            qh = q_ref[pl.ds(r0, QBLK), pl.ds(h * DH, DH)]
            s = lax.dot_general(qh, kbuf[...], (((1,), (1,)), ((), ())),
                                preferred_element_type=jnp.float32)
            row = lax.broadcasted_iota(jnp.int32, (QBLK, SKV), 0) + r0
            col = lax.broadcasted_iota(jnp.int32, (QBLK, SKV), 1)
            qblk_id = row // 64
            kblk_id = col // 64
            mask = ((qblk_id == kblk_id) | (kblk_id == 0)
                    | (lax.rem(qblk_id + kblk_id, 3) == 0))
            s = jnp.where(mask, s * SCALE, -1e9)
            m = jnp.max(s, axis=1, keepdims=True)
            w = jnp.exp(s - m)
            den = jnp.sum(w, axis=1, keepdims=True)
            c = lax.dot_general(w, vbuf[...], (((1,), (0,)), ((), ())),
                                preferred_element_type=jnp.float32)
            ctx[pl.ds(r0, QBLK), pl.ds(h * DH, DH)] = c / den
            return carry

        lax.fori_loop(0, H_LOC * N_QBLK, attn_step, 0)

        part[...] = lax.dot_general(ctx[...], wo_ref[...],
                                    (((1,), (0,)), ((), ())),
                                    preferred_element_type=jnp.float32)

        for s in range(N_DEV - 1):
            send_chunk = lax.rem(my + N_DEV - 1 - s, N_DEV)
            if s == 0:
                src = part.at[pl.ds(send_chunk * SQ_LOC, SQ_LOC), :]
            else:
                src = rs_send.at[s - 1]
            rdma = pltpu.make_async_remote_copy(
                src_ref=src,
                dst_ref=rs_recv.at[s],
                send_sem=rs_send_sems.at[s],
                recv_sem=rs_recv_sems.at[s],
                device_id=(right,),
                device_id_type=pl.DeviceIdType.MESH,
            )
            rdma.start()
            rdma.wait()
            recv_chunk = lax.rem(my + N_DEV - 2 - s, N_DEV)
            if s < N_DEV - 2:
                rs_send[s] = (rs_recv[s]
                              + part[pl.ds(recv_chunk * SQ_LOC, SQ_LOC), :])
            else:
                out_ref[0] = (rs_recv[s]
                              + part[pl.ds(my * SQ_LOC, SQ_LOC), :])

    return pl.pallas_call(
        body,
        out_shape=jax.ShapeDtypeStruct((1, SQ_LOC, D_MODEL), jnp.float32),
        in_specs=[
            pl.BlockSpec(memory_space=pltpu.VMEM),
            pl.BlockSpec(memory_space=pltpu.VMEM),
            pl.BlockSpec(memory_space=pl.ANY),
            pl.BlockSpec(memory_space=pl.ANY),
            pl.BlockSpec(memory_space=pltpu.VMEM),
        ],
        out_specs=pl.BlockSpec(memory_space=pltpu.VMEM),
        scratch_shapes=[
            pltpu.VMEM((N_DEV, SQ_LOC, D_MODEL), jnp.float32),
            pltpu.VMEM((SQ, D_HEADS), jnp.float32),
            pltpu.VMEM((SKV, DH), jnp.float32),
            pltpu.VMEM((SKV, DH), jnp.float32),
            pltpu.VMEM((SQ, D_HEADS), jnp.float32),
            pltpu.VMEM((SQ, D_MODEL), jnp.float32),
            pltpu.VMEM((N_DEV - 2, SQ_LOC, D_MODEL), jnp.float32),
            pltpu.VMEM((N_DEV - 1, SQ_LOC, D_MODEL), jnp.float32),
            pltpu.SemaphoreType.DMA((2,)),
            pltpu.SemaphoreType.DMA((N_DEV - 1,)),
            pltpu.SemaphoreType.DMA((N_DEV - 1,)),
            pltpu.SemaphoreType.DMA((N_DEV - 1,)),
            pltpu.SemaphoreType.DMA((N_DEV - 1,)),
        ],
        compiler_params=pltpu.CompilerParams(
            collective_id=0,
            vmem_limit_bytes=64 * 1024 * 1024,
        ),
    )(x2, Wq, K_ext, V_ext, Wo)


# device time: 131868 ns/iter; 1.9137x vs baseline; 1.9137x over previous
import jax
import jax.numpy as jnp
from jax import lax
from jax.experimental import pallas as pl
from jax.experimental.pallas import tpu as pltpu

N_DEV = 4
SQ_LOC = 256
D_MODEL = 1024
SKV = 4096
H_LOC = 8
DH = 128
SQ = SQ_LOC * N_DEV
D_HEADS = H_LOC * DH
SCALE = 0.08838834764831843
BLK = 64
N_QB = SQ // BLK

KEEP = [
    list(range(0, 64, 3)),
    [0] + list(range(2, 64, 3)),
    [0] + list(range(1, 64, 3)),
]
assert all(len(k) == 22 for k in KEEP)
NKB = 22
KV_LEN = NKB * BLK

GRP = [[qb for qb in range(N_QB) if qb % 3 == r] for r in range(3)]
GLEN = [len(g) * BLK for g in GRP]
GBASE = [0, GLEN[0], GLEN[0] + GLEN[1]]


def kernel(x, Wq, K_ext, V_ext, Wo):
    x2 = x.reshape(SQ_LOC, D_MODEL)

    def body(x_ref, wq_ref, k_hbm, v_hbm, wo_ref, out_ref,
             comm, qg, kg, vg, kd, vd, ctx, part, rs_send, rs_recv,
             kg_sem, vg_sem, kd_sem, vd_sem,
             ag_send_sems, ag_recv_sems, rs_send_sems, rs_recv_sems):
        my = lax.axis_index("i")
        left = lax.rem(my + N_DEV - 1, N_DEV)
        right = lax.rem(my + 1, N_DEV)
        h0 = my * H_LOC

        barrier = pltpu.get_barrier_semaphore()
        for nbr in (left, right):
            pl.semaphore_signal(barrier, inc=1, device_id=(nbr,),
                                device_id_type=pl.DeviceIdType.MESH)
        pl.semaphore_wait(barrier, 2)

        def kv_dma_descs(h, slot):
            hh = h0 + h
            descs = []
            for g in range(3):
                for i, kb in enumerate(KEEP[g]):
                    descs.append(pltpu.make_async_copy(
                        k_hbm.at[0, pl.ds(kb * BLK, BLK), hh, :],
                        kg.at[slot, g, pl.ds(i * BLK, BLK), :], kg_sem))
                    descs.append(pltpu.make_async_copy(
                        v_hbm.at[0, pl.ds(kb * BLK, BLK), hh, :],
                        vg.at[slot, g, pl.ds(i * BLK, BLK), :], vg_sem))
            for g in (1, 2):
                for t, qb in enumerate(GRP[g]):
                    descs.append(pltpu.make_async_copy(
                        k_hbm.at[0, pl.ds(qb * BLK, BLK), hh, :],
                        kd.at[slot, g - 1, pl.ds(t * BLK, BLK), :], kd_sem))
                    descs.append(pltpu.make_async_copy(
                        v_hbm.at[0, pl.ds(qb * BLK, BLK), hh, :],
                        vd.at[slot, g - 1, pl.ds(t * BLK, BLK), :], vd_sem))
            return descs

        for d in kv_dma_descs(0, 0):
            d.start()

        def store_q_chunk(qv, origin):
            for j in range(4):
                qb = 4 * origin + j
                r = lax.rem(qb, 3)
                t = qb // 3
                dest = (jnp.where(r == 0, 0,
                                  jnp.where(r == 1, GBASE[1], GBASE[2]))
                        + t * BLK)
                qg[pl.ds(dest, BLK), :] = qv[j * BLK:(j + 1) * BLK, :]

        comm[0] = x_ref[...]
        rdmas = []
        for hop in range(N_DEV - 1):
            rdmas.append(pltpu.make_async_remote_copy(
                src_ref=comm.at[hop],
                dst_ref=comm.at[hop + 1],
                send_sem=ag_send_sems.at[hop],
                recv_sem=ag_recv_sems.at[hop],
                device_id=(right,),
                device_id_type=pl.DeviceIdType.MESH,
            ))
        for hop in range(N_DEV - 1):
            rdmas[hop].start()
            src = x_ref if hop == 0 else comm.at[hop]
            origin = lax.rem(my + N_DEV - hop, N_DEV) if hop else my
            store_q_chunk(
                lax.dot_general(src[...], wq_ref[...],
                                (((1,), (0,)), ((), ())),
                                preferred_element_type=jnp.float32),
                origin)
            rdmas[hop].wait()
        store_q_chunk(
            lax.dot_general(comm[N_DEV - 1], wq_ref[...],
                            (((1,), (0,)), ((), ())),
                            preferred_element_type=jnp.float32),
            lax.rem(my + 1, N_DEV))

        def head_body(h, carry):
            slot = lax.rem(h, 2)
            for d in kv_dma_descs(h, slot):
                d.wait()

            @pl.when(h < H_LOC - 1)
            def _():
                for d in kv_dma_descs(h + 1, 1 - slot):
                    d.start()

            hc = pl.ds(h * DH, DH)
            qs = qg[pl.ds(GBASE[0], GLEN[0]), hc]
            sc = lax.dot_general(qs, kg[slot, 0],
                                 (((1,), (1,)), ((), ())),
                                 preferred_element_type=jnp.float32) * SCALE
            m = jnp.max(sc, axis=1, keepdims=True)
            w = jnp.exp(sc - m)
            den = jnp.sum(w, axis=1, keepdims=True)
            c = lax.dot_general(w, vg[slot, 0],
                                (((1,), (0,)), ((), ())),
                                preferred_element_type=jnp.float32) / den
            for t, qb in enumerate(GRP[0]):
                ctx[pl.ds(qb * BLK, BLK), hc] = c[t * BLK:(t + 1) * BLK, :]

            for r in (1, 2):
                L = GLEN[r]
                qs = qg[pl.ds(GBASE[r], L), hc]
                sc = lax.dot_general(qs, kg[slot, r],
                                     (((1,), (1,)), ((), ())),
                                     preferred_element_type=jnp.float32
                                     ) * SCALE
                sd = lax.dot_general(qs, kd[slot, r - 1],
                                     (((1,), (1,)), ((), ())),
                                     preferred_element_type=jnp.float32
                                     ) * SCALE
                ri = lax.broadcasted_iota(jnp.int32, (L, L), 0) // BLK
                ci = lax.broadcasted_iota(jnp.int32, (L, L), 1) // BLK
                sd = jnp.where(ri == ci, sd, -1e9)
                m = jnp.maximum(jnp.max(sc, axis=1, keepdims=True),
                                jnp.max(sd, axis=1, keepdims=True))
                wc = jnp.exp(sc - m)
                wd = jnp.exp(sd - m)
                den = (jnp.sum(wc, axis=1, keepdims=True)
                       + jnp.sum(wd, axis=1, keepdims=True))
                c = (lax.dot_general(wc, vg[slot, r],
                                     (((1,), (0,)), ((), ())),
                                     preferred_element_type=jnp.float32)
                     + lax.dot_general(wd, vd[slot, r - 1],
                                       (((1,), (0,)), ((), ())),
                                       preferred_element_type=jnp.float32)
                     ) / den
                for t, qb in enumerate(GRP[r]):
                    ctx[pl.ds(qb * BLK, BLK), hc] = c[t * BLK:(t + 1) * BLK, :]
            return carry

        lax.fori_loop(0, H_LOC, head_body, 0)

        part[...] = lax.dot_general(ctx[...], wo_ref[...],
                                    (((1,), (0,)), ((), ())),
                                    preferred_element_type=jnp.float32)

        for s in range(N_DEV - 1):
            send_chunk = lax.rem(my + N_DEV - 1 - s, N_DEV)
            if s == 0:
                src = part.at[pl.ds(send_chunk * SQ_LOC, SQ_LOC), :]
            else:
                src = rs_send.at[s - 1]
            rdma = pltpu.make_async_remote_copy(
                src_ref=src,
                dst_ref=rs_recv.at[s],
                send_sem=rs_send_sems.at[s],
                recv_sem=rs_recv_sems.at[s],
                device_id=(right,),
                device_id_type=pl.DeviceIdType.MESH,
            )
            rdma.start()
            rdma.wait()
            recv_chunk = lax.rem(my + N_DEV - 2 - s, N_DEV)
            if s < N_DEV - 2:
                rs_send[s] = (rs_recv[s]
                              + part[pl.ds(recv_chunk * SQ_LOC, SQ_LOC), :])
            else:
                out_ref[0] = (rs_recv[s]
                              + part[pl.ds(my * SQ_LOC, SQ_LOC), :])

    return pl.pallas_call(
        body,
        out_shape=jax.ShapeDtypeStruct((1, SQ_LOC, D_MODEL), jnp.float32),
        in_specs=[
            pl.BlockSpec(memory_space=pltpu.VMEM),
            pl.BlockSpec(memory_space=pltpu.VMEM),
            pl.BlockSpec(memory_space=pl.ANY),
            pl.BlockSpec(memory_space=pl.ANY),
            pl.BlockSpec(memory_space=pltpu.VMEM),
        ],
        out_specs=pl.BlockSpec(memory_space=pltpu.VMEM),
        scratch_shapes=[
            pltpu.VMEM((N_DEV, SQ_LOC, D_MODEL), jnp.float32),
            pltpu.VMEM((SQ, D_HEADS), jnp.float32),
            pltpu.VMEM((2, 3, KV_LEN, DH), jnp.float32),
            pltpu.VMEM((2, 3, KV_LEN, DH), jnp.float32),
            pltpu.VMEM((2, 2, GLEN[1], DH), jnp.float32),
            pltpu.VMEM((2, 2, GLEN[1], DH), jnp.float32),
            pltpu.VMEM((SQ, D_HEADS), jnp.float32),
            pltpu.VMEM((SQ, D_MODEL), jnp.float32),
            pltpu.VMEM((N_DEV - 2, SQ_LOC, D_MODEL), jnp.float32),
            pltpu.VMEM((N_DEV - 1, SQ_LOC, D_MODEL), jnp.float32),
            pltpu.SemaphoreType.DMA,
            pltpu.SemaphoreType.DMA,
            pltpu.SemaphoreType.DMA,
            pltpu.SemaphoreType.DMA,
            pltpu.SemaphoreType.DMA((N_DEV - 1,)),
            pltpu.SemaphoreType.DMA((N_DEV - 1,)),
            pltpu.SemaphoreType.DMA((N_DEV - 1,)),
            pltpu.SemaphoreType.DMA((N_DEV - 1,)),
        ],
        compiler_params=pltpu.CompilerParams(
            collective_id=0,
            vmem_limit_bytes=64 * 1024 * 1024,
        ),
    )(x2, Wq, K_ext, V_ext, Wo)


# device time: 131832 ns/iter; 1.9142x vs baseline; 1.0003x over previous
import jax
import jax.numpy as jnp
from jax import lax
from jax.experimental import pallas as pl
from jax.experimental.pallas import tpu as pltpu

N_DEV = 4
SQ_LOC = 256
D_MODEL = 1024
SKV = 4096
H_LOC = 8
DH = 128
SQ = SQ_LOC * N_DEV
D_HEADS = H_LOC * DH
SCALE = 0.08838834764831843
BLK = 64
N_QB = SQ // BLK

KEEP = [
    list(range(0, 64, 3)),
    [0] + list(range(2, 64, 3)),
    [0] + list(range(1, 64, 3)),
]
assert all(len(k) == 22 for k in KEEP)
NKB = 22
KV_LEN = NKB * BLK

GRP = [[qb for qb in range(N_QB) if qb % 3 == r] for r in range(3)]
GLEN = [len(g) * BLK for g in GRP]
GBASE = [0, GLEN[0], GLEN[0] + GLEN[1]]

F32 = jnp.float32
BF16 = jnp.bfloat16
DOT = (((1,), (0,)), ((), ()))
DOT_T = (((1,), (1,)), ((), ()))


def kernel(x, Wq, K_ext, V_ext, Wo):
    x2 = x.reshape(SQ_LOC, D_MODEL)

    def body(x_ref, wq_ref, k_hbm, v_hbm, wo_ref, out_ref,
             comm, qg, kg, vg, kd, vd, kgb, vgb, kdb, vdb,
             wqb, wob, ctx, rs_part, rs_send, rs_recv,
             kg_sem, vg_sem, kd_sem, vd_sem,
             ag_send_sems, ag_recv_sems, rs_send_sems, rs_recv_sems):
        my = lax.axis_index("i")
        left = lax.rem(my + N_DEV - 1, N_DEV)
        right = lax.rem(my + 1, N_DEV)
        h0 = my * H_LOC

        barrier = pltpu.get_barrier_semaphore()
        for nbr in (left, right):
            pl.semaphore_signal(barrier, inc=1, device_id=(nbr,),
                                device_id_type=pl.DeviceIdType.MESH)
        pl.semaphore_wait(barrier, 2)

        def kv_dma_descs(h, slot):
            hh = h0 + h
            descs = []
            for g in range(3):
                for i, kb in enumerate(KEEP[g]):
                    descs.append(pltpu.make_async_copy(
                        k_hbm.at[0, pl.ds(kb * BLK, BLK), hh, :],
                        kg.at[slot, g, pl.ds(i * BLK, BLK), :], kg_sem))
                    descs.append(pltpu.make_async_copy(
                        v_hbm.at[0, pl.ds(kb * BLK, BLK), hh, :],
                        vg.at[slot, g, pl.ds(i * BLK, BLK), :], vg_sem))
            for g in (1, 2):
                for t, qb in enumerate(GRP[g]):
                    descs.append(pltpu.make_async_copy(
                        k_hbm.at[0, pl.ds(qb * BLK, BLK), hh, :],
                        kd.at[slot, g - 1, pl.ds(t * BLK, BLK), :], kd_sem))
                    descs.append(pltpu.make_async_copy(
                        v_hbm.at[0, pl.ds(qb * BLK, BLK), hh, :],
                        vd.at[slot, g - 1, pl.ds(t * BLK, BLK), :], vd_sem))
            return descs

        for d in kv_dma_descs(0, 0):
            d.start()
        wqb[...] = wq_ref[...].astype(BF16)
        wob[...] = wo_ref[...].astype(BF16)

        def store_q_chunk(chunk, origin):
            qv = lax.dot_general(chunk.astype(BF16), wqb[...], DOT,
                                 preferred_element_type=F32).astype(BF16)
            for j in range(4):
                qb = 4 * origin + j
                r = lax.rem(qb, 3)
                t = qb // 3
                dest = (jnp.where(r == 0, 0,
                                  jnp.where(r == 1, GBASE[1], GBASE[2]))
                        + t * BLK)
                qg[pl.ds(dest, BLK), :] = qv[j * BLK:(j + 1) * BLK, :]

        comm[0] = x_ref[...]
        rdmas = []
        for hop in range(N_DEV - 1):
            rdmas.append(pltpu.make_async_remote_copy(
                src_ref=comm.at[hop],
                dst_ref=comm.at[hop + 1],
                send_sem=ag_send_sems.at[hop],
                recv_sem=ag_recv_sems.at[hop],
                device_id=(right,),
                device_id_type=pl.DeviceIdType.MESH,
            ))
        for hop in range(N_DEV - 1):
            rdmas[hop].start()
            if hop == 0:
                store_q_chunk(x_ref[...], my)
            else:
                store_q_chunk(comm[hop], lax.rem(my + N_DEV - hop, N_DEV))
            rdmas[hop].wait()
        store_q_chunk(comm[N_DEV - 1], lax.rem(my + 1, N_DEV))

        def head_body(h, carry):
            slot = lax.rem(h, 2)
            for d in kv_dma_descs(h, slot):
                d.wait()

            @pl.when(h < H_LOC - 1)
            def _():
                for d in kv_dma_descs(h + 1, 1 - slot):
                    d.start()

            kgb[...] = kg[slot].astype(BF16)
            vgb[...] = vg[slot].astype(BF16)
            kdb[...] = kd[slot].astype(BF16)
            vdb[...] = vd[slot].astype(BF16)

            hc = pl.ds(h * DH, DH)
            qs = qg[pl.ds(GBASE[0], GLEN[0]), hc]
            sc = lax.dot_general(qs, kgb[0], DOT_T,
                                 preferred_element_type=F32) * SCALE
            m = jnp.max(sc, axis=1, keepdims=True)
            w = jnp.exp(sc - m)
            den = jnp.sum(w, axis=1, keepdims=True)
            c = lax.dot_general(w.astype(BF16), vgb[0], DOT,
                                preferred_element_type=F32) / den
            for t, qb in enumerate(GRP[0]):
                ctx[pl.ds(qb * BLK, BLK), hc] = (
                    c[t * BLK:(t + 1) * BLK, :].astype(BF16))

            for r in (1, 2):
                L = GLEN[r]
                qs = qg[pl.ds(GBASE[r], L), hc]
                sc = lax.dot_general(qs, kgb[r], DOT_T,
                                     preferred_element_type=F32) * SCALE
                sd = lax.dot_general(qs, kdb[r - 1], DOT_T,
                                     preferred_element_type=F32) * SCALE
                ri = lax.broadcasted_iota(jnp.int32, (L, L), 0) // BLK
                ci = lax.broadcasted_iota(jnp.int32, (L, L), 1) // BLK
                sd = jnp.where(ri == ci, sd, -1e9)
                m = jnp.maximum(jnp.max(sc, axis=1, keepdims=True),
                                jnp.max(sd, axis=1, keepdims=True))
                wc = jnp.exp(sc - m)
                wd = jnp.exp(sd - m)
                den = (jnp.sum(wc, axis=1, keepdims=True)
                       + jnp.sum(wd, axis=1, keepdims=True))
                c = (lax.dot_general(wc.astype(BF16), vgb[r], DOT,
                                     preferred_element_type=F32)
                     + lax.dot_general(wd.astype(BF16), vdb[r - 1], DOT,
                                       preferred_element_type=F32)) / den
                for t, qb in enumerate(GRP[r]):
                    ctx[pl.ds(qb * BLK, BLK), hc] = (
                        c[t * BLK:(t + 1) * BLK, :].astype(BF16))
            return carry

        lax.fori_loop(0, H_LOC, head_body, 0)

        def part_chunk(c):
            return lax.dot_general(ctx[pl.ds(c * SQ_LOC, SQ_LOC), :],
                                   wob[...], DOT,
                                   preferred_element_type=F32)

        rs_part[0] = part_chunk(lax.rem(my + N_DEV - 1, N_DEV))
        for s in range(N_DEV - 1):
            src = rs_part.at[0] if s == 0 else rs_send.at[s - 1]
            rdma = pltpu.make_async_remote_copy(
                src_ref=src,
                dst_ref=rs_recv.at[s],
                send_sem=rs_send_sems.at[s],
                recv_sem=rs_recv_sems.at[s],
                device_id=(right,),
                device_id_type=pl.DeviceIdType.MESH,
            )
            rdma.start()
            rs_part[s + 1] = part_chunk(lax.rem(my + N_DEV - 2 - s, N_DEV))
            rdma.wait()
            if s < N_DEV - 2:
                rs_send[s] = rs_recv[s] + rs_part[s + 1]
            else:
                out_ref[0] = rs_recv[s] + rs_part[s + 1]

    return pl.pallas_call(
        body,
        out_shape=jax.ShapeDtypeStruct((1, SQ_LOC, D_MODEL), jnp.float32),
        in_specs=[
            pl.BlockSpec(memory_space=pltpu.VMEM),
            pl.BlockSpec(memory_space=pltpu.VMEM),
            pl.BlockSpec(memory_space=pl.ANY),
            pl.BlockSpec(memory_space=pl.ANY),
            pl.BlockSpec(memory_space=pltpu.VMEM),
        ],
        out_specs=pl.BlockSpec(memory_space=pltpu.VMEM),
        scratch_shapes=[
            pltpu.VMEM((N_DEV, SQ_LOC, D_MODEL), F32),
            pltpu.VMEM((SQ, D_HEADS), BF16),
            pltpu.VMEM((2, 3, KV_LEN, DH), F32),
            pltpu.VMEM((2, 3, KV_LEN, DH), F32),
            pltpu.VMEM((2, 2, GLEN[1], DH), F32),
            pltpu.VMEM((2, 2, GLEN[1], DH), F32),
            pltpu.VMEM((3, KV_LEN, DH), BF16),
            pltpu.VMEM((3, KV_LEN, DH), BF16),
            pltpu.VMEM((2, GLEN[1], DH), BF16),
            pltpu.VMEM((2, GLEN[1], DH), BF16),
            pltpu.VMEM((D_MODEL, D_HEADS), BF16),
            pltpu.VMEM((D_HEADS, D_MODEL), BF16),
            pltpu.VMEM((SQ, D_HEADS), BF16),
            pltpu.VMEM((N_DEV, SQ_LOC, D_MODEL), F32),
            pltpu.VMEM((N_DEV - 2, SQ_LOC, D_MODEL), F32),
            pltpu.VMEM((N_DEV - 1, SQ_LOC, D_MODEL), F32),
            pltpu.SemaphoreType.DMA,
            pltpu.SemaphoreType.DMA,
            pltpu.SemaphoreType.DMA,
            pltpu.SemaphoreType.DMA,
            pltpu.SemaphoreType.DMA((N_DEV - 1,)),
            pltpu.SemaphoreType.DMA((N_DEV - 1,)),
            pltpu.SemaphoreType.DMA((N_DEV - 1,)),
            pltpu.SemaphoreType.DMA((N_DEV - 1,)),
        ],
        compiler_params=pltpu.CompilerParams(
            collective_id=0,
            vmem_limit_bytes=64 * 1024 * 1024,
        ),
    )(x2, Wq, K_ext, V_ext, Wo)


# device time: 99174 ns/iter; 2.5446x vs baseline; 1.3293x over previous
import os

import jax
import jax.numpy as jnp
from jax import lax
from jax.experimental import pallas as pl
from jax.experimental.pallas import tpu as pltpu

ABLATE = os.environ.get("ABLATE", "")

N_DEV = 4
SQ_LOC = 256
HSQ = SQ_LOC // 2
D_MODEL = 1024
SKV = 4096
H_LOC = 8
DH = 128
SQ = SQ_LOC * N_DEV
D_HEADS = H_LOC * DH
SCALE = 0.08838834764831843
BLK = 64
N_QB = SQ // BLK

KEEP = [
    list(range(0, 64, 3)),
    [0] + list(range(2, 64, 3)),
    [0] + list(range(1, 64, 3)),
]
NKB = 22
KV_LEN = NKB * BLK

GRP = [[qb for qb in range(N_QB) if qb % 3 == r] for r in range(3)]
GLEN = [len(g) * BLK for g in GRP]
GBASE = [0, GLEN[0], GLEN[0] + GLEN[1]]

F32 = jnp.float32
BF16 = jnp.bfloat16
DOT = (((1,), (0,)), ((), ()))
DOT_T = (((1,), (1,)), ((), ()))


def kernel(x, Wq, K_ext, V_ext, Wo):
    x2 = x.reshape(SQ_LOC, D_MODEL)

    def body(x_ref, wq_ref, k_hbm, v_hbm, wo_ref, out_ref,
             comm_r, comm_l, qg, kg, vg, kd, vd, kgb, vgb, kdb, vdb,
             wqb, wob, ctx, rs_part, rs_send_r, rs_send_l,
             rs_recv_r, rs_recv_l,
             kg_sem, vg_sem, kd_sem, vd_sem,
             agr_s, agr_r, agl_s, agl_r,
             rsr_s, rsr_r, rsl_s, rsl_r):
        my = lax.axis_index("i")
        left = lax.rem(my + N_DEV - 1, N_DEV)
        right = lax.rem(my + 1, N_DEV)
        h0 = my * H_LOC

        barrier = pltpu.get_barrier_semaphore()
        for nbr in (left, right):
            pl.semaphore_signal(barrier, inc=1, device_id=(nbr,),
                                device_id_type=pl.DeviceIdType.MESH)
        pl.semaphore_wait(barrier, 2)

        def kv_dma_descs(h, slot):
            hh = h0 + h
            descs = []
            for g in range(3):
                for i, kb in enumerate(KEEP[g]):
                    descs.append(pltpu.make_async_copy(
                        k_hbm.at[0, pl.ds(kb * BLK, BLK), hh, :],
                        kg.at[slot, g, pl.ds(i * BLK, BLK), :],
                        kg_sem.at[slot]))
                    descs.append(pltpu.make_async_copy(
                        v_hbm.at[0, pl.ds(kb * BLK, BLK), hh, :],
                        vg.at[slot, g, pl.ds(i * BLK, BLK), :],
                        vg_sem.at[slot]))
            for g in (1, 2):
                for t, qb in enumerate(GRP[g]):
                    descs.append(pltpu.make_async_copy(
                        k_hbm.at[0, pl.ds(qb * BLK, BLK), hh, :],
                        kd.at[slot, g - 1, pl.ds(t * BLK, BLK), :],
                        kd_sem.at[slot]))
                    descs.append(pltpu.make_async_copy(
                        v_hbm.at[0, pl.ds(qb * BLK, BLK), hh, :],
                        vd.at[slot, g - 1, pl.ds(t * BLK, BLK), :],
                        vd_sem.at[slot]))
            return descs

        if not ABLATE:
            for d in kv_dma_descs(0, 0):
                d.start()
        wqb[...] = wq_ref[...].astype(BF16)
        wob[...] = wo_ref[...].astype(BF16)

        def store_q_half(half_val, origin, half):
            qv = lax.dot_general(half_val.astype(BF16), wqb[...], DOT,
                                 preferred_element_type=F32).astype(BF16)
            for j in range(2):
                qb = 4 * origin + 2 * half + j
                r = lax.rem(qb, 3)
                t = qb // 3
                dest = (jnp.where(r == 0, 0,
                                  jnp.where(r == 1, GBASE[1], GBASE[2]))
                        + t * BLK)
                qg[pl.ds(dest, BLK), :] = qv[j * BLK:(j + 1) * BLK, :]

        comm_r[0] = x_ref[pl.ds(0, HSQ), :]
        comm_l[0] = x_ref[pl.ds(HSQ, HSQ), :]
        ag = []
        for hop in range(N_DEV - 1):
            ag.append((
                pltpu.make_async_remote_copy(
                    src_ref=comm_r.at[hop], dst_ref=comm_r.at[hop + 1],
                    send_sem=agr_s.at[hop], recv_sem=agr_r.at[hop],
                    device_id=(right,),
                    device_id_type=pl.DeviceIdType.MESH),
                pltpu.make_async_remote_copy(
                    src_ref=comm_l.at[hop], dst_ref=comm_l.at[hop + 1],
                    send_sem=agl_s.at[hop], recv_sem=agl_r.at[hop],
                    device_id=(left,),
                    device_id_type=pl.DeviceIdType.MESH),
            ))
        for hop in range(N_DEV - 1):
            ag[hop][0].start()
            ag[hop][1].start()
            if hop == 0:
                store_q_half(x_ref[pl.ds(0, HSQ), :], my, 0)
                store_q_half(x_ref[pl.ds(HSQ, HSQ), :], my, 1)
            else:
                store_q_half(comm_r[hop],
                             lax.rem(my + N_DEV - hop, N_DEV), 0)
                store_q_half(comm_l[hop], lax.rem(my + hop, N_DEV), 1)
            ag[hop][0].wait()
            ag[hop][1].wait()
        store_q_half(comm_r[N_DEV - 1], lax.rem(my + 1, N_DEV), 0)
        store_q_half(comm_l[N_DEV - 1], lax.rem(my + N_DEV - 1, N_DEV), 1)

        def head_body(h, carry):
            slot = lax.rem(h, 2)
            if ABLATE != "nodma":
                @pl.when(h < H_LOC - 1)
                def _():
                    for d in kv_dma_descs(h + 1, 1 - slot):
                        d.start()

                for d in kv_dma_descs(h, slot):
                    d.wait()

            kgb[...] = kg[slot].astype(BF16)
            vgb[...] = vg[slot].astype(BF16)
            kdb[...] = kd[slot].astype(BF16)
            vdb[...] = vd[slot].astype(BF16)

            hc = pl.ds(h * DH, DH)
            qs = qg[pl.ds(GBASE[0], GLEN[0]), hc]
            sc = lax.dot_general(qs, kgb[0], DOT_T,
                                 preferred_element_type=F32) * SCALE
            m = jnp.max(sc, axis=1, keepdims=True)
            w = jnp.exp(sc - m)
            den = jnp.sum(w, axis=1, keepdims=True)
            c = lax.dot_general(w.astype(BF16), vgb[0], DOT,
                                preferred_element_type=F32) / den
            for t, qb in enumerate(GRP[0]):
                ctx[pl.ds(qb * BLK, BLK), hc] = (
                    c[t * BLK:(t + 1) * BLK, :].astype(BF16))

            for r in (1, 2):
                L = GLEN[r]
                qs = qg[pl.ds(GBASE[r], L), hc]
                sc = lax.dot_general(qs, kgb[r], DOT_T,
                                     preferred_element_type=F32) * SCALE
                sd = lax.dot_general(qs, kdb[r - 1], DOT_T,
                                     preferred_element_type=F32) * SCALE
                ri = lax.broadcasted_iota(jnp.int32, (L, L), 0) // BLK
                ci = lax.broadcasted_iota(jnp.int32, (L, L), 1) // BLK
                sd = jnp.where(ri == ci, sd, -1e9)
                m = jnp.maximum(jnp.max(sc, axis=1, keepdims=True),
                                jnp.max(sd, axis=1, keepdims=True))
                wc = jnp.exp(sc - m)
                wd = jnp.exp(sd - m)
                den = (jnp.sum(wc, axis=1, keepdims=True)
                       + jnp.sum(wd, axis=1, keepdims=True))
                c = (lax.dot_general(wc.astype(BF16), vgb[r], DOT,
                                     preferred_element_type=F32)
                     + lax.dot_general(wd.astype(BF16), vdb[r - 1], DOT,
                                       preferred_element_type=F32)) / den
                for t, qb in enumerate(GRP[r]):
                    ctx[pl.ds(qb * BLK, BLK), hc] = (
                        c[t * BLK:(t + 1) * BLK, :].astype(BF16))
            return carry

        if ABLATE != "noattn":
            lax.fori_loop(0, H_LOC, head_body, 0)

        def part_chunk(c):
            return lax.dot_general(ctx[pl.ds(c * SQ_LOC, SQ_LOC), :],
                                   wob[...], DOT,
                                   preferred_element_type=F32)

        rs_part[0] = part_chunk(lax.rem(my + 3, N_DEV))
        rs_part[1] = part_chunk(lax.rem(my + 1, N_DEV))
        for s in range(N_DEV - 1):
            if s == 0:
                src_r = rs_part.at[0, pl.ds(0, HSQ), :]
                src_l = rs_part.at[1, pl.ds(HSQ, HSQ), :]
            else:
                src_r = rs_send_r.at[s - 1]
                src_l = rs_send_l.at[s - 1]
            rd_r = pltpu.make_async_remote_copy(
                src_ref=src_r, dst_ref=rs_recv_r.at[s],
                send_sem=rsr_s.at[s], recv_sem=rsr_r.at[s],
                device_id=(right,), device_id_type=pl.DeviceIdType.MESH)
            rd_l = pltpu.make_async_remote_copy(
                src_ref=src_l, dst_ref=rs_recv_l.at[s],
                send_sem=rsl_s.at[s], recv_sem=rsl_r.at[s],
                device_id=(left,), device_id_type=pl.DeviceIdType.MESH)
            rd_r.start()
            rd_l.start()
            if s == 0:
                rs_part[2] = part_chunk(lax.rem(my + 2, N_DEV))
            elif s == 1:
                rs_part[3] = part_chunk(my)
            rd_r.wait()
            rd_l.wait()
            if s == 0:
                rs_send_r[0] = rs_recv_r[0] + rs_part[2, pl.ds(0, HSQ), :]
                rs_send_l[0] = rs_recv_l[0] + rs_part[2, pl.ds(HSQ, HSQ), :]
            elif s == 1:
                rs_send_r[1] = rs_recv_r[1] + rs_part[1, pl.ds(0, HSQ), :]
                rs_send_l[1] = rs_recv_l[1] + rs_part[0, pl.ds(HSQ, HSQ), :]
            else:
                out_ref[0, pl.ds(0, HSQ), :] = (
                    rs_recv_r[2] + rs_part[3, pl.ds(0, HSQ), :])
                out_ref[0, pl.ds(HSQ, HSQ), :] = (
                    rs_recv_l[2] + rs_part[3, pl.ds(HSQ, HSQ), :])

    return pl.pallas_call(
        body,
        out_shape=jax.ShapeDtypeStruct((1, SQ_LOC, D_MODEL), jnp.float32),
        in_specs=[
            pl.BlockSpec(memory_space=pltpu.VMEM),
            pl.BlockSpec(memory_space=pltpu.VMEM),
            pl.BlockSpec(memory_space=pl.ANY),
            pl.BlockSpec(memory_space=pl.ANY),
            pl.BlockSpec(memory_space=pltpu.VMEM),
        ],
        out_specs=pl.BlockSpec(memory_space=pltpu.VMEM),
        scratch_shapes=[
            pltpu.VMEM((N_DEV, HSQ, D_MODEL), F32),
            pltpu.VMEM((N_DEV, HSQ, D_MODEL), F32),
            pltpu.VMEM((SQ, D_HEADS), BF16),
            pltpu.VMEM((2, 3, KV_LEN, DH), F32),
            pltpu.VMEM((2, 3, KV_LEN, DH), F32),
            pltpu.VMEM((2, 2, GLEN[1], DH), F32),
            pltpu.VMEM((2, 2, GLEN[1], DH), F32),
            pltpu.VMEM((3, KV_LEN, DH), BF16),
            pltpu.VMEM((3, KV_LEN, DH), BF16),
            pltpu.VMEM((2, GLEN[1], DH), BF16),
            pltpu.VMEM((2, GLEN[1], DH), BF16),
            pltpu.VMEM((D_MODEL, D_HEADS), BF16),
            pltpu.VMEM((D_HEADS, D_MODEL), BF16),
            pltpu.VMEM((SQ, D_HEADS), BF16),
            pltpu.VMEM((N_DEV, SQ_LOC, D_MODEL), F32),
            pltpu.VMEM((N_DEV - 2, HSQ, D_MODEL), F32),
            pltpu.VMEM((N_DEV - 2, HSQ, D_MODEL), F32),
            pltpu.VMEM((N_DEV - 1, HSQ, D_MODEL), F32),
            pltpu.VMEM((N_DEV - 1, HSQ, D_MODEL), F32),
            pltpu.SemaphoreType.DMA((2,)),
            pltpu.SemaphoreType.DMA((2,)),
            pltpu.SemaphoreType.DMA((2,)),
            pltpu.SemaphoreType.DMA((2,)),
            pltpu.SemaphoreType.DMA((N_DEV - 1,)),
            pltpu.SemaphoreType.DMA((N_DEV - 1,)),
            pltpu.SemaphoreType.DMA((N_DEV - 1,)),
            pltpu.SemaphoreType.DMA((N_DEV - 1,)),
            pltpu.SemaphoreType.DMA((N_DEV - 1,)),
            pltpu.SemaphoreType.DMA((N_DEV - 1,)),
            pltpu.SemaphoreType.DMA((N_DEV - 1,)),
            pltpu.SemaphoreType.DMA((N_DEV - 1,)),
        ],
        compiler_params=pltpu.CompilerParams(
            collective_id=0,
            vmem_limit_bytes=64 * 1024 * 1024,
        ),
    )(x2, Wq, K_ext, V_ext, Wo)


# device time: 82225 ns/iter; 3.0691x vs baseline; 1.2061x over previous
import os

import jax
import jax.numpy as jnp
from jax import lax
from jax.experimental import pallas as pl
from jax.experimental.pallas import tpu as pltpu

ABLATE = os.environ.get("ABLATE", "")

N_DEV = 4
SQ_LOC = 256
HSQ = SQ_LOC // 2
D_MODEL = 1024
SKV = 4096
H_LOC = 8
DH = 128
SQ = SQ_LOC * N_DEV
D_HEADS = H_LOC * DH
SCALE = 0.08838834764831843
BLK = 64
N_QB = SQ // BLK

KEEP = [
    list(range(0, 64, 3)),
    [0] + list(range(2, 64, 3)),
    [0] + list(range(1, 64, 3)),
]
NKB = 22
KV_LEN = NKB * BLK

GRP = [[qb for qb in range(N_QB) if qb % 3 == r] for r in range(3)]
GLEN = [len(g) * BLK for g in GRP]
GBASE = [0, GLEN[0], GLEN[0] + GLEN[1]]

F32 = jnp.float32
BF16 = jnp.bfloat16
DOT = (((1,), (0,)), ((), ()))
DOT_T = (((1,), (1,)), ((), ()))


def kernel(x, Wq, K_ext, V_ext, Wo):
    x2 = x.reshape(SQ_LOC, D_MODEL)

    def body(x_ref, wq_ref, k_hbm, v_hbm, wo_ref, out_ref,
             comm_r, comm_l, qg, kg, vg, kd, vd, kgb, vgb, kdb, vdb,
             wqb, wob, ctx, rs_part, rs_stage_r, rs_stage_l,
             rs_send_r, rs_send_l, rs_recv_r, rs_recv_l,
             kg_sem, vg_sem, kd_sem, vd_sem,
             agr_s, agr_r, agl_s, agl_r,
             rsr_s, rsr_r, rsl_s, rsl_r):
        my = lax.axis_index("i")
        left = lax.rem(my + N_DEV - 1, N_DEV)
        right = lax.rem(my + 1, N_DEV)
        h0 = my * H_LOC

        barrier = pltpu.get_barrier_semaphore()
        for nbr in (left, right):
            pl.semaphore_signal(barrier, inc=1, device_id=(nbr,),
                                device_id_type=pl.DeviceIdType.MESH)
        pl.semaphore_wait(barrier, 2)

        def kv_dma_descs(h, slot):
            hh = h0 + h
            descs = []
            for g in range(3):
                for i, kb in enumerate(KEEP[g]):
                    descs.append(pltpu.make_async_copy(
                        k_hbm.at[0, pl.ds(kb * BLK, BLK), hh, :],
                        kg.at[slot, g, pl.ds(i * BLK, BLK), :],
                        kg_sem.at[slot]))
                    descs.append(pltpu.make_async_copy(
                        v_hbm.at[0, pl.ds(kb * BLK, BLK), hh, :],
                        vg.at[slot, g, pl.ds(i * BLK, BLK), :],
                        vg_sem.at[slot]))
            for g in (1, 2):
                for t, qb in enumerate(GRP[g]):
                    descs.append(pltpu.make_async_copy(
                        k_hbm.at[0, pl.ds(qb * BLK, BLK), hh, :],
                        kd.at[slot, g - 1, pl.ds(t * BLK, BLK), :],
                        kd_sem.at[slot]))
                    descs.append(pltpu.make_async_copy(
                        v_hbm.at[0, pl.ds(qb * BLK, BLK), hh, :],
                        vd.at[slot, g - 1, pl.ds(t * BLK, BLK), :],
                        vd_sem.at[slot]))
            return descs

        if not ABLATE:
            for d in kv_dma_descs(0, 0):
                d.start()
        wqb[...] = wq_ref[...].astype(BF16)
        wob[...] = wo_ref[...].astype(BF16)

        def store_q_half(half_val, origin, half):
            qv = lax.dot_general(half_val, wqb[...], DOT,
                                 preferred_element_type=F32).astype(BF16)
            for j in range(2):
                qb = 4 * origin + 2 * half + j
                r = lax.rem(qb, 3)
                t = qb // 3
                dest = (jnp.where(r == 0, 0,
                                  jnp.where(r == 1, GBASE[1], GBASE[2]))
                        + t * BLK)
                qg[pl.ds(dest, BLK), :] = qv[j * BLK:(j + 1) * BLK, :]

        comm_r[0] = x_ref[pl.ds(0, HSQ), :].astype(BF16)
        comm_l[0] = x_ref[pl.ds(HSQ, HSQ), :].astype(BF16)
        ag = []
        for hop in range(N_DEV - 1):
            ag.append((
                pltpu.make_async_remote_copy(
                    src_ref=comm_r.at[hop], dst_ref=comm_r.at[hop + 1],
                    send_sem=agr_s.at[hop], recv_sem=agr_r.at[hop],
                    device_id=(right,),
                    device_id_type=pl.DeviceIdType.MESH),
                pltpu.make_async_remote_copy(
                    src_ref=comm_l.at[hop], dst_ref=comm_l.at[hop + 1],
                    send_sem=agl_s.at[hop], recv_sem=agl_r.at[hop],
                    device_id=(left,),
                    device_id_type=pl.DeviceIdType.MESH),
            ))
        for hop in range(N_DEV - 1):
            ag[hop][0].start()
            ag[hop][1].start()
            if hop == 0:
                store_q_half(comm_r[0], my, 0)
                store_q_half(comm_l[0], my, 1)
            else:
                store_q_half(comm_r[hop],
                             lax.rem(my + N_DEV - hop, N_DEV), 0)
                store_q_half(comm_l[hop], lax.rem(my + hop, N_DEV), 1)
            ag[hop][0].wait()
            ag[hop][1].wait()
        store_q_half(comm_r[N_DEV - 1], lax.rem(my + 1, N_DEV), 0)
        store_q_half(comm_l[N_DEV - 1], lax.rem(my + N_DEV - 1, N_DEV), 1)

        def head_body(h, carry):
            slot = lax.rem(h, 2)
            if ABLATE != "nodma":
                @pl.when(h < H_LOC - 1)
                def _():
                    for d in kv_dma_descs(h + 1, 1 - slot):
                        d.start()

                for d in kv_dma_descs(h, slot):
                    d.wait()

            kgb[...] = kg[slot].astype(BF16)
            vgb[...] = vg[slot].astype(BF16)
            kdb[...] = kd[slot].astype(BF16)
            vdb[...] = vd[slot].astype(BF16)

            hc = pl.ds(h * DH, DH)
            qs = qg[pl.ds(GBASE[0], GLEN[0]), hc]
            sc = lax.dot_general(qs, kgb[0], DOT_T,
                                 preferred_element_type=F32) * SCALE
            m = jnp.max(sc, axis=1, keepdims=True)
            w = jnp.exp(sc - m)
            den = jnp.sum(w, axis=1, keepdims=True)
            c = lax.dot_general(w.astype(BF16), vgb[0], DOT,
                                preferred_element_type=F32) / den
            for t, qb in enumerate(GRP[0]):
                ctx[pl.ds(qb * BLK, BLK), hc] = (
                    c[t * BLK:(t + 1) * BLK, :].astype(BF16))

            for r in (1, 2):
                L = GLEN[r]
                qs = qg[pl.ds(GBASE[r], L), hc]
                sc = lax.dot_general(qs, kgb[r], DOT_T,
                                     preferred_element_type=F32) * SCALE
                sd = lax.dot_general(qs, kdb[r - 1], DOT_T,
                                     preferred_element_type=F32) * SCALE
                ri = lax.broadcasted_iota(jnp.int32, (L, L), 0) // BLK
                ci = lax.broadcasted_iota(jnp.int32, (L, L), 1) // BLK
                sd = jnp.where(ri == ci, sd, -1e9)
                m = jnp.maximum(jnp.max(sc, axis=1, keepdims=True),
                                jnp.max(sd, axis=1, keepdims=True))
                wc = jnp.exp(sc - m)
                wd = jnp.exp(sd - m)
                den = (jnp.sum(wc, axis=1, keepdims=True)
                       + jnp.sum(wd, axis=1, keepdims=True))
                c = (lax.dot_general(wc.astype(BF16), vgb[r], DOT,
                                     preferred_element_type=F32)
                     + lax.dot_general(wd.astype(BF16), vdb[r - 1], DOT,
                                       preferred_element_type=F32)) / den
                for t, qb in enumerate(GRP[r]):
                    ctx[pl.ds(qb * BLK, BLK), hc] = (
                        c[t * BLK:(t + 1) * BLK, :].astype(BF16))
            return carry

        if ABLATE != "noattn":
            lax.fori_loop(0, H_LOC, head_body, 0)

        def part_chunk(c):
            return lax.dot_general(ctx[pl.ds(c * SQ_LOC, SQ_LOC), :],
                                   wob[...], DOT,
                                   preferred_element_type=F32)

        rs_part[0] = part_chunk(lax.rem(my + 3, N_DEV))
        rs_part[1] = part_chunk(lax.rem(my + 1, N_DEV))
        rs_stage_r[...] = rs_part[0, pl.ds(0, HSQ), :].astype(BF16)
        rs_stage_l[...] = rs_part[1, pl.ds(HSQ, HSQ), :].astype(BF16)
        for s in range(N_DEV - 1):
            if s == 0:
                src_r = rs_stage_r
                src_l = rs_stage_l
            else:
                src_r = rs_send_r.at[s - 1]
                src_l = rs_send_l.at[s - 1]
            rd_r = pltpu.make_async_remote_copy(
                src_ref=src_r, dst_ref=rs_recv_r.at[s],
                send_sem=rsr_s.at[s], recv_sem=rsr_r.at[s],
                device_id=(right,), device_id_type=pl.DeviceIdType.MESH)
            rd_l = pltpu.make_async_remote_copy(
                src_ref=src_l, dst_ref=rs_recv_l.at[s],
                send_sem=rsl_s.at[s], recv_sem=rsl_r.at[s],
                device_id=(left,), device_id_type=pl.DeviceIdType.MESH)
            rd_r.start()
            rd_l.start()
            if s == 0:
                rs_part[2] = part_chunk(lax.rem(my + 2, N_DEV))
            elif s == 1:
                rs_part[3] = part_chunk(my)
            rd_r.wait()
            rd_l.wait()
            if s == 0:
                rs_send_r[0] = (rs_recv_r[0].astype(F32)
                                + rs_part[2, pl.ds(0, HSQ), :]).astype(BF16)
                rs_send_l[0] = (rs_recv_l[0].astype(F32)
                                + rs_part[2, pl.ds(HSQ, HSQ), :]).astype(BF16)
            elif s == 1:
                rs_send_r[1] = (rs_recv_r[1].astype(F32)
                                + rs_part[1, pl.ds(0, HSQ), :]).astype(BF16)
                rs_send_l[1] = (rs_recv_l[1].astype(F32)
                                + rs_part[0, pl.ds(HSQ, HSQ), :]).astype(BF16)
            else:
                out_ref[0, pl.ds(0, HSQ), :] = (
                    rs_recv_r[2].astype(F32) + rs_part[3, pl.ds(0, HSQ), :])
                out_ref[0, pl.ds(HSQ, HSQ), :] = (
                    rs_recv_l[2].astype(F32)
                    + rs_part[3, pl.ds(HSQ, HSQ), :])

    return pl.pallas_call(
        body,
        out_shape=jax.ShapeDtypeStruct((1, SQ_LOC, D_MODEL), jnp.float32),
        in_specs=[
            pl.BlockSpec(memory_space=pltpu.VMEM),
            pl.BlockSpec(memory_space=pltpu.VMEM),
            pl.BlockSpec(memory_space=pl.ANY),
            pl.BlockSpec(memory_space=pl.ANY),
            pl.BlockSpec(memory_space=pltpu.VMEM),
        ],
        out_specs=pl.BlockSpec(memory_space=pltpu.VMEM),
        scratch_shapes=[
            pltpu.VMEM((N_DEV, HSQ, D_MODEL), BF16),
            pltpu.VMEM((N_DEV, HSQ, D_MODEL), BF16),
            pltpu.VMEM((SQ, D_HEADS), BF16),
            pltpu.VMEM((2, 3, KV_LEN, DH), F32),
            pltpu.VMEM((2, 3, KV_LEN, DH), F32),
            pltpu.VMEM((2, 2, GLEN[1], DH), F32),
            pltpu.VMEM((2, 2, GLEN[1], DH), F32),
            pltpu.VMEM((3, KV_LEN, DH), BF16),
            pltpu.VMEM((3, KV_LEN, DH), BF16),
            pltpu.VMEM((2, GLEN[1], DH), BF16),
            pltpu.VMEM((2, GLEN[1], DH), BF16),
            pltpu.VMEM((D_MODEL, D_HEADS), BF16),
            pltpu.VMEM((D_HEADS, D_MODEL), BF16),
            pltpu.VMEM((SQ, D_HEADS), BF16),
            pltpu.VMEM((N_DEV, SQ_LOC, D_MODEL), F32),
            pltpu.VMEM((HSQ, D_MODEL), BF16),
            pltpu.VMEM((HSQ, D_MODEL), BF16),
            pltpu.VMEM((N_DEV - 2, HSQ, D_MODEL), BF16),
            pltpu.VMEM((N_DEV - 2, HSQ, D_MODEL), BF16),
            pltpu.VMEM((N_DEV - 1, HSQ, D_MODEL), BF16),
            pltpu.VMEM((N_DEV - 1, HSQ, D_MODEL), BF16),
            pltpu.SemaphoreType.DMA((2,)),
            pltpu.SemaphoreType.DMA((2,)),
            pltpu.SemaphoreType.DMA((2,)),
            pltpu.SemaphoreType.DMA((2,)),
            pltpu.SemaphoreType.DMA((N_DEV - 1,)),
            pltpu.SemaphoreType.DMA((N_DEV - 1,)),
            pltpu.SemaphoreType.DMA((N_DEV - 1,)),
            pltpu.SemaphoreType.DMA((N_DEV - 1,)),
            pltpu.SemaphoreType.DMA((N_DEV - 1,)),
            pltpu.SemaphoreType.DMA((N_DEV - 1,)),
            pltpu.SemaphoreType.DMA((N_DEV - 1,)),
            pltpu.SemaphoreType.DMA((N_DEV - 1,)),
        ],
        compiler_params=pltpu.CompilerParams(
            collective_id=0,
            vmem_limit_bytes=64 * 1024 * 1024,
        ),
    )(x2, Wq, K_ext, V_ext, Wo)
